# baseline (speedup 1.0000x reference)
import functools

import jax
import jax.numpy as jnp
from jax import lax
from jax.experimental import pallas as pl
from jax.experimental.pallas import tpu as pltpu
from jax.experimental.pallas import tpu_sc as plsc

BATCH = 16384
EMB = 64
NC = 2
NS = 16
NW = NC * NS
B_PER_W = BATCH // NW
IDX_CHUNK = 128
NK = B_PER_W // IDX_CHUNK


def _sc_gather_one(idx, tab_p):
    nb = idx.shape[0]
    b_per_w = nb // NW
    nk = b_per_w // IDX_CHUNK
    mesh = plsc.VectorSubcoreMesh(core_axis_name="c", subcore_axis_name="s")

    @functools.partial(
        pl.kernel,
        mesh=mesh,
        out_type=jax.ShapeDtypeStruct((nb, 2 * EMB), jnp.int32),
        scratch_types=[
            pltpu.VMEM((nk, IDX_CHUNK), jnp.int32),
            pltpu.VMEM((nk, IDX_CHUNK, 2 * EMB), jnp.int32),
            pltpu.SemaphoreType.DMA,
        ],
    )
    def gather_k(idx_hbm, tab_hbm, out_hbm, idx_v, rows_v, sem):
        wid = lax.axis_index("s") * NC + lax.axis_index("c")
        base = wid * b_per_w
        for k in range(nk):
            pltpu.sync_copy(
                idx_hbm.at[pl.ds(base + k * IDX_CHUNK, IDX_CHUNK)],
                idx_v.at[k])
        copies = [
            pltpu.async_copy(tab_hbm.at[idx_v.at[k]], rows_v.at[k], sem)
            for k in range(nk)
        ]
        for c in copies:
            c.wait()
        for k in range(nk):
            pltpu.sync_copy(
                rows_v.at[k],
                out_hbm.at[pl.ds(base + k * IDX_CHUNK, IDX_CHUNK)])

    return gather_k(idx, tab_p)


PACK_BN = 57344


def _pack_body(t_ref, o_ref):
    ft = jnp.float32
    ident = (lax.broadcasted_iota(jnp.int32, (EMB, EMB), 0)
             == lax.broadcasted_iota(jnp.int32, (EMB, EMB), 1)
             ).astype(jnp.bfloat16)
    q = PACK_BN // 4
    dn = (((0,), (0,)), ((), ()))
    bits = []
    for k in range(4):
        xk = lax.dot_general(
            t_ref[:, k * q:(k + 1) * q].astype(jnp.bfloat16), ident, dn,
            preferred_element_type=ft)
        bits.append(lax.bitcast_convert_type(xk, jnp.int32))
    lo_mask = jnp.int32(0xffff)
    hi_mask = jnp.int32(-65536)
    left = (lax.shift_right_logical(bits[0], 16) & lo_mask) | (bits[1] & hi_mask)
    right = (lax.shift_right_logical(bits[2], 16) & lo_mask) | (bits[3] & hi_mask)
    o_ref[...] = jnp.concatenate([left, right], axis=1)


def _tc_pack(tab_t, n_rows):
    bn = PACK_BN
    grid = (n_rows + bn - 1) // bn
    return pl.pallas_call(
        _pack_body,
        grid=(grid,),
        in_specs=[pl.BlockSpec((EMB, bn), lambda i: (0, i))],
        out_specs=pl.BlockSpec((bn // 4, 2 * EMB), lambda i: (i, 0)),
        out_shape=jax.ShapeDtypeStruct((grid * (bn // 4), 2 * EMB),
                                       jnp.int32),
        compiler_params=pltpu.CompilerParams(vmem_limit_bytes=100 * 2**20),
    )(tab_t)


def _unpack_select(x_i32, pbm, phm):
    f32 = jnp.float32
    hi_mask = jnp.int32(-65536)
    left = x_i32[:, :EMB]
    right = x_i32[:, EMB:]
    lo_l = lax.bitcast_convert_type(lax.shift_left(left, 16), f32)
    hi_l = lax.bitcast_convert_type(left & hi_mask, f32)
    lo_r = lax.bitcast_convert_type(lax.shift_left(right, 16), f32)
    hi_r = lax.bitcast_convert_type(right & hi_mask, f32)
    ll = jnp.where(pbm, hi_l, lo_l)
    rr = jnp.where(pbm, hi_r, lo_r)
    return jnp.where(phm, rr, ll)


def _mlp_body(xu_ref, xm_ref, pp_ref, w1u_ref, w1m_ref, b1_ref,
              w2_ref, b2_ref, w3_ref, b3_ref, o_ref):
    f32 = jnp.float32
    bf = jnp.bfloat16
    dn_t = (((1,), (1,)), ((), ()))
    dn_k1 = (((1,), (0,)), ((), ()))
    sel = (lax.broadcasted_iota(jnp.int32, (4, 4 * EMB), 0)
           == lax.broadcasted_iota(jnp.int32, (4, 4 * EMB), 1) // EMB
           ).astype(f32)
    pall = lax.dot_general(pp_ref[...], sel, dn_k1,
                           preferred_element_type=f32) > 0.5
    u = _unpack_select(xu_ref[...], pall[:, :EMB],
                       pall[:, EMB:2 * EMB]).astype(bf)
    m = _unpack_select(xm_ref[...], pall[:, 2 * EMB:3 * EMB],
                       pall[:, 3 * EMB:]).astype(bf)
    x = lax.dot_general(u, w1u_ref[...].astype(bf), dn_t,
                        preferred_element_type=f32)
    x = x + lax.dot_general(m, w1m_ref[...].astype(bf), dn_t,
                            preferred_element_type=f32)
    x = jnp.maximum(x + b1_ref[...], 0.0).astype(bf)
    y = lax.dot_general(x, w2_ref[...].astype(bf), dn_t,
                        preferred_element_type=f32)
    y = jnp.maximum(y + b2_ref[...], 0.0)
    z = jnp.sum(y * w3_ref[...], axis=1, keepdims=True)
    o_ref[...] = z + b3_ref[0, 0]


def _tc_mlp(xu, xm, pp, W1, b1, W2, b2, W3, b3, bs=4096):
    nb = xu.shape[0]
    W1u = W1[:, :EMB]
    W1m = W1[:, EMB:]
    grid = nb // bs
    full = lambda i: (0, 0)
    row = lambda i: (i, 0)
    out = pl.pallas_call(
        _mlp_body,
        grid=(grid,),
        in_specs=[
            pl.BlockSpec((bs, 2 * EMB), row),
            pl.BlockSpec((bs, 2 * EMB), row),
            pl.BlockSpec((bs, 4), row),
            pl.BlockSpec(W1u.shape, full),
            pl.BlockSpec(W1m.shape, full),
            pl.BlockSpec((1, 128), full),
            pl.BlockSpec(W2.shape, full),
            pl.BlockSpec((1, 64), full),
            pl.BlockSpec(W3.shape, full),
            pl.BlockSpec((1, 1), full),
        ],
        out_specs=pl.BlockSpec((bs, 1), row),
        out_shape=jax.ShapeDtypeStruct((nb, 1), jnp.float32),
        compiler_params=pltpu.CompilerParams(vmem_limit_bytes=100 * 2**20),
    )(xu, xm, pp, W1u, W1m, b1.reshape(1, 128), W2, b2.reshape(1, 64),
      W3, b3.reshape(1, 1))
    return out


def kernel(user_idx, movie_idx, user_emb, movie_emb, W1, b1, W2, b2, W3, b3):
    ui = user_idx.astype(jnp.int32)
    mi = movie_idx.astype(jnp.int32)
    bn = PACK_BN
    q = bn // 4
    uc = ui % bn
    mc = mi % bn
    u_half = (ui // bn) * q + uc % q
    m_half = (mi // bn) * q + mc % q
    uq = uc // q
    mq = mc // q
    pp = jnp.stack([(uq & 1).astype(jnp.float32),
                    (uq >> 1).astype(jnp.float32),
                    (mq & 1).astype(jnp.float32),
                    (mq >> 1).astype(jnp.float32)], axis=1)
    mtab_p = _tc_pack(movie_emb.T, movie_emb.shape[0])
    xm = _sc_gather_one(m_half, mtab_p)
    utab_p = _tc_pack(user_emb.T, user_emb.shape[0])
    h = BATCH // 2
    xu0 = _sc_gather_one(u_half[:h], utab_p)
    xu1 = _sc_gather_one(u_half[h:], utab_p)
    o0 = _tc_mlp(xu0, xm[:h], pp[:h], W1, b1, W2, b2, W3, b3)
    o1 = _tc_mlp(xu1, xm[h:], pp[h:], W1, b1, W2, b2, W3, b3)
    return jnp.concatenate([o0, o1], axis=0)


# speedup vs baseline: 1.0807x; 1.0807x over previous
import functools

import jax
import jax.numpy as jnp
from jax import lax
from jax.experimental import pallas as pl
from jax.experimental.pallas import tpu as pltpu
from jax.experimental.pallas import tpu_sc as plsc

BATCH = 16384
EMB = 64
NC = 2
NS = 16
NW = NC * NS
B_PER_W = BATCH // NW
IDX_CHUNK = 128
NK = B_PER_W // IDX_CHUNK


def _sc_gather_one(idx, tab_p):
    mesh = plsc.VectorSubcoreMesh(core_axis_name="c", subcore_axis_name="s")

    @functools.partial(
        pl.kernel,
        mesh=mesh,
        out_type=jax.ShapeDtypeStruct((BATCH, 2 * EMB), jnp.int32),
        scratch_types=[
            pltpu.VMEM((NK, IDX_CHUNK), jnp.int32),
            pltpu.VMEM((NK, IDX_CHUNK, 2 * EMB), jnp.int32),
            pltpu.SemaphoreType.DMA,
        ],
    )
    def gather_k(idx_hbm, tab_hbm, out_hbm, idx_v, rows_v, sem):
        wid = lax.axis_index("s") * NC + lax.axis_index("c")
        base = wid * B_PER_W
        for k in range(NK):
            pltpu.sync_copy(
                idx_hbm.at[pl.ds(base + k * IDX_CHUNK, IDX_CHUNK)],
                idx_v.at[k])
        copies = [
            pltpu.async_copy(tab_hbm.at[idx_v.at[k]], rows_v.at[k], sem)
            for k in range(NK)
        ]
        for c in copies:
            c.wait()
        for k in range(NK):
            pltpu.sync_copy(
                rows_v.at[k],
                out_hbm.at[pl.ds(base + k * IDX_CHUNK, IDX_CHUNK)])

    return gather_k(idx, tab_p)


PACK_BN = 57344


def _pack_body(t_ref, o_ref):
    ft = jnp.float32
    ident = (lax.broadcasted_iota(jnp.int32, (EMB, EMB), 0)
             == lax.broadcasted_iota(jnp.int32, (EMB, EMB), 1)
             ).astype(jnp.bfloat16)
    q = PACK_BN // 4
    dn = (((0,), (0,)), ((), ()))
    bits = []
    for k in range(4):
        xk = lax.dot_general(
            t_ref[:, k * q:(k + 1) * q].astype(jnp.bfloat16), ident, dn,
            preferred_element_type=ft)
        bits.append(lax.bitcast_convert_type(xk, jnp.int32))
    lo_mask = jnp.int32(0xffff)
    hi_mask = jnp.int32(-65536)
    left = (lax.shift_right_logical(bits[0], 16) & lo_mask) | (bits[1] & hi_mask)
    right = (lax.shift_right_logical(bits[2], 16) & lo_mask) | (bits[3] & hi_mask)
    o_ref[...] = jnp.concatenate([left, right], axis=1)


def _tc_pack(tab_t, n_rows):
    bn = PACK_BN
    grid = (n_rows + bn - 1) // bn
    return pl.pallas_call(
        _pack_body,
        grid=(grid,),
        in_specs=[pl.BlockSpec((EMB, bn), lambda i: (0, i))],
        out_specs=pl.BlockSpec((bn // 4, 2 * EMB), lambda i: (i, 0)),
        out_shape=jax.ShapeDtypeStruct((grid * (bn // 4), 2 * EMB),
                                       jnp.int32),
        compiler_params=pltpu.CompilerParams(vmem_limit_bytes=100 * 2**20),
    )(tab_t)


def _unpack_select(x_i32, pbm, phm):
    f32 = jnp.float32
    hi_mask = jnp.int32(-65536)
    left = x_i32[:, :EMB]
    right = x_i32[:, EMB:]
    lo_l = lax.bitcast_convert_type(lax.shift_left(left, 16), f32)
    hi_l = lax.bitcast_convert_type(left & hi_mask, f32)
    lo_r = lax.bitcast_convert_type(lax.shift_left(right, 16), f32)
    hi_r = lax.bitcast_convert_type(right & hi_mask, f32)
    ll = jnp.where(pbm, hi_l, lo_l)
    rr = jnp.where(pbm, hi_r, lo_r)
    return jnp.where(phm, rr, ll)


def _mlp_body(xu_ref, xm_ref, pp_ref, w1u_ref, w1m_ref, b1_ref,
              w2_ref, b2_ref, w3_ref, b3_ref, o_ref):
    f32 = jnp.float32
    bf = jnp.bfloat16
    dn_t = (((1,), (1,)), ((), ()))
    dn_k1 = (((1,), (0,)), ((), ()))
    sel = (lax.broadcasted_iota(jnp.int32, (4, 4 * EMB), 0)
           == lax.broadcasted_iota(jnp.int32, (4, 4 * EMB), 1) // EMB
           ).astype(f32)
    pall = lax.dot_general(pp_ref[...], sel, dn_k1,
                           preferred_element_type=f32) > 0.5
    u = _unpack_select(xu_ref[...], pall[:, :EMB],
                       pall[:, EMB:2 * EMB]).astype(bf)
    m = _unpack_select(xm_ref[...], pall[:, 2 * EMB:3 * EMB],
                       pall[:, 3 * EMB:]).astype(bf)
    x = lax.dot_general(u, w1u_ref[...].astype(bf), dn_t,
                        preferred_element_type=f32)
    x = x + lax.dot_general(m, w1m_ref[...].astype(bf), dn_t,
                            preferred_element_type=f32)
    x = jnp.maximum(x + b1_ref[...], 0.0).astype(bf)
    y = lax.dot_general(x, w2_ref[...].astype(bf), dn_t,
                        preferred_element_type=f32)
    y = jnp.maximum(y + b2_ref[...], 0.0)
    z = jnp.sum(y * w3_ref[...], axis=1, keepdims=True)
    o_ref[...] = z + b3_ref[0, 0]


def _tc_mlp(xu, xm, pp, W1, b1, W2, b2, W3, b3, bs=4096):
    W1u = W1[:, :EMB]
    W1m = W1[:, EMB:]
    grid = BATCH // bs
    full = lambda i: (0, 0)
    row = lambda i: (i, 0)
    out = pl.pallas_call(
        _mlp_body,
        grid=(grid,),
        in_specs=[
            pl.BlockSpec((bs, 2 * EMB), row),
            pl.BlockSpec((bs, 2 * EMB), row),
            pl.BlockSpec((bs, 4), row),
            pl.BlockSpec(W1u.shape, full),
            pl.BlockSpec(W1m.shape, full),
            pl.BlockSpec((1, 128), full),
            pl.BlockSpec(W2.shape, full),
            pl.BlockSpec((1, 64), full),
            pl.BlockSpec(W3.shape, full),
            pl.BlockSpec((1, 1), full),
        ],
        out_specs=pl.BlockSpec((bs, 1), row),
        out_shape=jax.ShapeDtypeStruct((BATCH, 1), jnp.float32),
        compiler_params=pltpu.CompilerParams(vmem_limit_bytes=100 * 2**20),
    )(xu, xm, pp, W1u, W1m, b1.reshape(1, 128), W2, b2.reshape(1, 64),
      W3, b3.reshape(1, 1))
    return out


def kernel(user_idx, movie_idx, user_emb, movie_emb, W1, b1, W2, b2, W3, b3):
    ui = user_idx.astype(jnp.int32)
    mi = movie_idx.astype(jnp.int32)
    bn = PACK_BN
    q = bn // 4
    uc = ui % bn
    mc = mi % bn
    u_half = (ui // bn) * q + uc % q
    m_half = (mi // bn) * q + mc % q
    uq = uc // q
    mq = mc // q
    pp = jnp.stack([(uq & 1).astype(jnp.float32),
                    (uq >> 1).astype(jnp.float32),
                    (mq & 1).astype(jnp.float32),
                    (mq >> 1).astype(jnp.float32)], axis=1)
    mtab_p = _tc_pack(movie_emb.T, movie_emb.shape[0])
    xm = _sc_gather_one(m_half, mtab_p)
    utab_p = _tc_pack(user_emb.T, user_emb.shape[0])
    xu = _sc_gather_one(u_half, utab_p)
    return _tc_mlp(xu, xm, pp, W1, b1, W2, b2, W3, b3)
